# baseline (speedup 1.0000x reference)
import jax
import jax.numpy as jnp
from jax.experimental import pallas as pl

TOP_K = 8
NUM_EXPERTS = 64
HIDDEN = 2048
BLOCK = 2048
CHUNK = 256
_IDX_MASK = NUM_EXPERTS - 1


def _router_body(hs_ref, wt_ref, probs_t_ref, scores_t_ref, idx_t_ref):
    wt = wt_ref[...]
    for c in range(BLOCK // CHUNK):
        rows = pl.ds(c * CHUNK, CHUNK)
        x = hs_ref[rows, :]
        logits = jax.lax.dot_general(
            x, wt, (((1,), (0,)), ((), ())),
            preferred_element_type=jnp.float32,
        )
        m = jnp.max(logits, axis=-1, keepdims=True)
        e = jnp.exp(logits - m)
        s = jnp.sum(e, axis=-1, keepdims=True)
        pn = e / s

        p = pn.T
        probs_t_ref[:, rows] = p
        iota = jax.lax.broadcasted_iota(jnp.int32, p.shape, 0)
        pwork = p
        vals = []
        inds = []
        for _ in range(TOP_K):
            mx = jnp.max(pwork, axis=0, keepdims=True)
            eq = pwork == mx
            ind = jnp.min(jnp.where(eq, iota, NUM_EXPERTS), axis=0, keepdims=True)
            vals.append(mx)
            inds.append(ind)
            pwork = jnp.where(eq & (iota == ind), -1.0, pwork)
        v = jnp.concatenate(vals, axis=0)
        idx = jnp.concatenate(inds, axis=0)
        sc = v / jnp.sum(v, axis=0, keepdims=True)
        scores_t_ref[:, rows] = sc
        idx_t_ref[:, rows] = idx


@jax.jit
def kernel(hidden_states, W):
    hs = hidden_states.reshape(-1, HIDDEN)
    n = hs.shape[0]
    wt = W.T
    grid = (n // BLOCK,)
    probs_t, scores_t, idx_t = pl.pallas_call(
        _router_body,
        grid=grid,
        in_specs=[
            pl.BlockSpec((BLOCK, HIDDEN), lambda i: (i, 0)),
            pl.BlockSpec((HIDDEN, NUM_EXPERTS), lambda i: (0, 0)),
        ],
        out_specs=[
            pl.BlockSpec((NUM_EXPERTS, BLOCK), lambda i: (0, i)),
            pl.BlockSpec((TOP_K, BLOCK), lambda i: (0, i)),
            pl.BlockSpec((TOP_K, BLOCK), lambda i: (0, i)),
        ],
        out_shape=[
            jax.ShapeDtypeStruct((NUM_EXPERTS, n), jnp.float32),
            jax.ShapeDtypeStruct((TOP_K, n), jnp.float32),
            jax.ShapeDtypeStruct((TOP_K, n), jnp.int32),
        ],
    )(hs, wt)
    return (probs_t.T, scores_t.T, idx_t.T)


# speedup vs baseline: 1.0002x; 1.0002x over previous
import jax
import jax.numpy as jnp
from jax.experimental import pallas as pl

TOP_K = 8
NUM_EXPERTS = 64
HIDDEN = 2048
BLOCK = 2048
CHUNK = 256


def _router_body(hs_ref, wt_ref, probs_t_ref, scores_t_ref, idx_t_ref):
    wt = wt_ref[...]
    for c in range(BLOCK // CHUNK):
        rows = pl.ds(c * CHUNK, CHUNK)
        x = hs_ref[rows, :]
        logits = jax.lax.dot_general(
            x, wt, (((1,), (0,)), ((), ())),
            preferred_element_type=jnp.float32,
        )
        m = jnp.max(logits, axis=-1, keepdims=True)
        e = jnp.exp(logits - m)
        s = jnp.sum(e, axis=-1, keepdims=True)
        pn = e / s

        p = pn.T
        probs_t_ref[:, rows] = p
        iota = jax.lax.broadcasted_iota(jnp.int32, p.shape, 0)
        pwork = p
        vals = []
        inds = []
        for _ in range(TOP_K):
            mx = jnp.max(pwork, axis=0, keepdims=True)
            eq = pwork == mx
            ind = jnp.min(jnp.where(eq, iota, NUM_EXPERTS), axis=0, keepdims=True)
            vals.append(mx)
            inds.append(ind)
            pwork = jnp.where(eq & (iota == ind), -1.0, pwork)
        v = jnp.concatenate(vals, axis=0)
        idx = jnp.concatenate(inds, axis=0)
        sc = v / jnp.sum(v, axis=0, keepdims=True)
        scores_t_ref[:, rows] = sc
        idx_t_ref[:, rows] = idx


@jax.jit
def kernel(hidden_states, W):
    hs = hidden_states.reshape(-1, HIDDEN)
    n = hs.shape[0]
    wt = W.T
    grid = (n // BLOCK,)
    probs_t, scores_t, idx_t = pl.pallas_call(
        _router_body,
        grid=grid,
        in_specs=[
            pl.BlockSpec((BLOCK, HIDDEN), lambda i: (i, 0)),
            pl.BlockSpec((HIDDEN, NUM_EXPERTS), lambda i: (0, 0)),
        ],
        out_specs=[
            pl.BlockSpec((NUM_EXPERTS, BLOCK), lambda i: (0, i)),
            pl.BlockSpec((TOP_K, BLOCK), lambda i: (0, i)),
            pl.BlockSpec((TOP_K, BLOCK), lambda i: (0, i)),
        ],
        out_shape=[
            jax.ShapeDtypeStruct((NUM_EXPERTS, n), jnp.float32),
            jax.ShapeDtypeStruct((TOP_K, n), jnp.float32),
            jax.ShapeDtypeStruct((TOP_K, n), jnp.int32),
        ],
    )(hs, wt)
    return (probs_t.T, scores_t.T, idx_t.T)
